# baseline (speedup 1.0000x reference)
import functools

import jax
import jax.numpy as jnp
from jax import lax
from jax.experimental import pallas as pl
from jax.experimental.pallas import tpu as pltpu
from jax.experimental.pallas import tpu_sc as plsc

T, D, F, E, K = 2048, 1024, 4096, 8, 2

BLK_R = 128
PAD_U = BLK_R
R_PAD = T * K + E * PAD_U

NC, NS = 2, 16
NW = NC * NS
TPW = T // NW
CCH = 32
LANES = 16


def _router_body(x_ref, wr_ref, e1_ref, e2_ref, g1_ref, g2_ref):
    logits = jnp.dot(x_ref[...], wr_ref[...],
                     preferred_element_type=jnp.float32)
    cols = lax.broadcasted_iota(jnp.int32, logits.shape, 1)
    l1 = jnp.max(logits, axis=1, keepdims=True)
    i1 = jnp.min(jnp.where(logits == l1, cols, E), axis=1, keepdims=True)
    masked = jnp.where(cols == i1, -jnp.inf, logits)
    l2 = jnp.max(masked, axis=1, keepdims=True)
    i2 = jnp.min(jnp.where(masked == l2, cols, E), axis=1, keepdims=True)
    g1 = jax.nn.sigmoid(l1 - l2)
    e1_ref[...] = i1
    e2_ref[...] = i2
    g1_ref[...] = g1
    g2_ref[...] = 1.0 - g1


def _router(x, Wr):
    return pl.pallas_call(
        _router_body,
        out_shape=(
            jax.ShapeDtypeStruct((T, 1), jnp.int32),
            jax.ShapeDtypeStruct((T, 1), jnp.int32),
            jax.ShapeDtypeStruct((T, 1), jnp.float32),
            jax.ShapeDtypeStruct((T, 1), jnp.float32),
        ),
    )(x, Wr)


def _route_metadata(e1, e2, g1, g2):
    flat_e = jnp.concatenate([e1, e2], axis=1).reshape(-1)
    flat_g = jnp.concatenate([g1, g2], axis=1).reshape(-1)
    onehot = (flat_e[:, None] == jnp.arange(E, dtype=jnp.int32)[None, :])
    onehot = onehot.astype(jnp.int32)
    counts = jnp.sum(onehot, axis=0)
    padded = ((counts + PAD_U - 1) // PAD_U) * PAD_U
    start = jnp.concatenate([jnp.zeros((1,), jnp.int32),
                             jnp.cumsum(padded)[:-1].astype(jnp.int32)])
    rank = jnp.cumsum(onehot, axis=0) - onehot
    rank_a = jnp.sum(rank * onehot, axis=1)
    pos = start[flat_e] + rank_a
    enblk = (padded // PAD_U).astype(jnp.int32)
    pos_kt = pos.reshape(T, K).T.reshape(-1)
    gates_kt = flat_g.reshape(T, K).T.reshape(-1)
    return pos_kt, gates_kt, start, enblk


def _sc_dispatch_body(x_hbm, pos_hbm, g_hbm, xs_hbm, rg_hbm,
                      i0_v, i1_v, rows_v, g0_v, g1_v, s0, s1, s2, s3):
    wid = lax.axis_index("c") * NS + lax.axis_index("s")
    base = wid * TPW
    pltpu.sync_copy(pos_hbm.at[pl.ds(base, TPW)], i0_v)
    pltpu.sync_copy(pos_hbm.at[pl.ds(T + base, TPW)], i1_v)
    pltpu.sync_copy(x_hbm.at[pl.ds(base, TPW)], rows_v)
    pltpu.sync_copy(g_hbm.at[pl.ds(base, TPW)], g0_v)
    pltpu.sync_copy(g_hbm.at[pl.ds(T + base, TPW)], g1_v)
    cp0 = pltpu.async_copy(rows_v, xs_hbm.at[i0_v], s0)
    cp1 = pltpu.async_copy(rows_v, xs_hbm.at[i1_v], s1)
    cp2 = pltpu.async_copy(g0_v, rg_hbm.at[i0_v], s2)
    cp3 = pltpu.async_copy(g1_v, rg_hbm.at[i1_v], s3)
    cp0.wait()
    cp1.wait()
    cp2.wait()
    cp3.wait()


def _sc_dispatch(x, pos_kt, gates_kt):
    mesh = plsc.VectorSubcoreMesh(core_axis_name="c", subcore_axis_name="s")
    fn = functools.partial(
        pl.kernel,
        mesh=mesh,
        out_type=(
            jax.ShapeDtypeStruct((R_PAD, D), jnp.float32),
            jax.ShapeDtypeStruct((R_PAD,), jnp.float32),
        ),
        scratch_types=[
            pltpu.VMEM((TPW,), jnp.int32),
            pltpu.VMEM((TPW,), jnp.int32),
            pltpu.VMEM((TPW, D), jnp.float32),
            pltpu.VMEM((TPW,), jnp.float32),
            pltpu.VMEM((TPW,), jnp.float32),
            pltpu.SemaphoreType.DMA,
            pltpu.SemaphoreType.DMA,
            pltpu.SemaphoreType.DMA,
            pltpu.SemaphoreType.DMA,
        ],
    )(_sc_dispatch_body)
    return fn(x, pos_kt, gates_kt)


BLK_F = 512
NF = F // BLK_F


def _gemm_body(estart_ref, enblk_ref, xs_ref, w1_ref, w2_ref, g_ref, out_ref):
    e = pl.program_id(0)
    f = pl.program_id(1)
    s = estart_ref[e]
    n = enblk_ref[e]
    w1 = w1_ref[...]
    w2 = w2_ref[...]

    def blk(i, _):
        base = pl.multiple_of(s + i * PAD_U, PAD_U)
        rr = [pl.ds(base, BLK_R)]
        hs = [lax.dot_general(xs_ref[r, :], w1, (((1,), (1,)), ((), ())),
                              preferred_element_type=jnp.float32)
              for r in rr]
        ps = [lax.dot_general(h * jax.nn.sigmoid(h),
                              w2, (((1,), (1,)), ((), ())),
                              preferred_element_type=jnp.float32)
              for h in hs]

        @pl.when(f == 0)
        def _():
            for r, p in zip(rr, ps):
                out_ref[r, :] = p

        @pl.when((f > 0) & (f < NF - 1))
        def _():
            for r, p in zip(rr, ps):
                out_ref[r, :] = out_ref[r, :] + p

        @pl.when(f == NF - 1)
        def _():
            for r, p in zip(rr, ps):
                out_ref[r, :] = (out_ref[r, :] + p) * g_ref[r, :]

        return 0

    lax.fori_loop(0, n, blk, 0)


def _grouped_gemm(estart, enblk, xs, W1, W2, row_gate):
    grid_spec = pltpu.PrefetchScalarGridSpec(
        num_scalar_prefetch=2,
        grid=(E, NF),
        in_specs=[
            pl.BlockSpec((R_PAD, D), lambda e, f, es, en: (0, 0)),
            pl.BlockSpec((BLK_F, D), lambda e, f, es, en: (e * NF + f, 0)),
            pl.BlockSpec((D, BLK_F), lambda e, f, es, en: (e, f)),
            pl.BlockSpec((R_PAD, 1), lambda e, f, es, en: (0, 0)),
        ],
        out_specs=pl.BlockSpec((R_PAD, D), lambda e, f, es, en: (0, 0)),
    )
    return pl.pallas_call(
        _gemm_body,
        grid_spec=grid_spec,
        out_shape=jax.ShapeDtypeStruct((R_PAD, D), jnp.float32),
        compiler_params=pltpu.CompilerParams(
            dimension_semantics=("arbitrary", "arbitrary"),
            vmem_limit_bytes=63 * 1024 * 1024),
    )(estart, enblk, xs, W1.reshape(E * F, D), W2.reshape(E * D, F),
      row_gate[:, None])


def _sc_combine_body(ys_hbm, pos_hbm, out_hbm, i0_v, i1_v, r0_v, r1_v, sem):
    wid = lax.axis_index("c") * NS + lax.axis_index("s")
    base = wid * TPW
    for c in range(TPW // CCH):
        off = base + c * CCH
        pltpu.sync_copy(pos_hbm.at[pl.ds(off, CCH)], i0_v)
        pltpu.sync_copy(pos_hbm.at[pl.ds(T + off, CCH)], i1_v)
        cp0 = pltpu.async_copy(ys_hbm.at[i0_v], r0_v, sem)
        cp1 = pltpu.async_copy(ys_hbm.at[i1_v], r1_v, sem)
        cp0.wait()
        cp1.wait()

        def body(i, _):
            for j in range(D // LANES):
                sl = pl.ds(j * LANES, LANES)
                r0_v[i, sl] = r0_v[i, sl] + r1_v[i, sl]
            return 0

        lax.fori_loop(0, CCH, body, 0)
        pltpu.sync_copy(r0_v, out_hbm.at[pl.ds(off, CCH)])


def _sc_combine(ys, pos_kt):
    mesh = plsc.VectorSubcoreMesh(core_axis_name="c", subcore_axis_name="s")
    fn = functools.partial(
        pl.kernel,
        mesh=mesh,
        out_type=jax.ShapeDtypeStruct((T, D), jnp.float32),
        scratch_types=[
            pltpu.VMEM((CCH,), jnp.int32),
            pltpu.VMEM((CCH,), jnp.int32),
            pltpu.VMEM((CCH, D), jnp.float32),
            pltpu.VMEM((CCH, D), jnp.float32),
            pltpu.SemaphoreType.DMA,
        ],
    )(_sc_combine_body)
    return fn(ys, pos_kt)


def kernel(x, Wr, W1, W2):
    e1, e2, g1, g2 = _router(x, Wr)
    pos_kt, gates_kt, estart, enblk = _route_metadata(e1, e2, g1, g2)
    xs, row_gate = _sc_dispatch(x, pos_kt, gates_kt)
    ys = _grouped_gemm(estart, enblk, xs, W1, W2, row_gate)
    return _sc_combine(ys, pos_kt)


# speedup vs baseline: 1.0567x; 1.0567x over previous
import functools

import jax
import jax.numpy as jnp
from jax import lax
from jax.experimental import pallas as pl
from jax.experimental.pallas import tpu as pltpu
from jax.experimental.pallas import tpu_sc as plsc

T, D, F, E, K = 2048, 1024, 4096, 8, 2

BLK_R = 128
PAD_U = BLK_R
R_PAD = T * K + E * PAD_U

NC, NS = 2, 16
NW = NC * NS
TPW = T // NW
CCH = 32
LANES = 16


def _router_body(x_ref, wr_ref, e1_ref, e2_ref, g1_ref, g2_ref):
    logits = jnp.dot(x_ref[...], wr_ref[...],
                     preferred_element_type=jnp.float32)
    cols = lax.broadcasted_iota(jnp.int32, logits.shape, 1)
    l1 = jnp.max(logits, axis=1, keepdims=True)
    i1 = jnp.min(jnp.where(logits == l1, cols, E), axis=1, keepdims=True)
    masked = jnp.where(cols == i1, -jnp.inf, logits)
    l2 = jnp.max(masked, axis=1, keepdims=True)
    i2 = jnp.min(jnp.where(masked == l2, cols, E), axis=1, keepdims=True)
    g1 = jax.nn.sigmoid(l1 - l2)
    e1_ref[...] = i1
    e2_ref[...] = i2
    g1_ref[...] = g1
    g2_ref[...] = 1.0 - g1


def _router(x, Wr):
    return pl.pallas_call(
        _router_body,
        out_shape=(
            jax.ShapeDtypeStruct((T, 1), jnp.int32),
            jax.ShapeDtypeStruct((T, 1), jnp.int32),
            jax.ShapeDtypeStruct((T, 1), jnp.float32),
            jax.ShapeDtypeStruct((T, 1), jnp.float32),
        ),
    )(x, Wr)


def _route_metadata(e1, e2, g1, g2):
    flat_e = jnp.concatenate([e1, e2], axis=1).reshape(-1)
    flat_g = jnp.concatenate([g1, g2], axis=1).reshape(-1)
    onehot = (flat_e[:, None] == jnp.arange(E, dtype=jnp.int32)[None, :])
    onehot = onehot.astype(jnp.int32)
    counts = jnp.sum(onehot, axis=0)
    padded = ((counts + PAD_U - 1) // PAD_U) * PAD_U
    start = jnp.concatenate([jnp.zeros((1,), jnp.int32),
                             jnp.cumsum(padded)[:-1].astype(jnp.int32)])
    rank = jnp.cumsum(onehot, axis=0) - onehot
    rank_a = jnp.sum(rank * onehot, axis=1)
    pos = start[flat_e] + rank_a
    row_gate = jnp.zeros((R_PAD,), jnp.float32).at[pos].set(flat_g)
    enblk = (padded // PAD_U).astype(jnp.int32)
    pos_kt = pos.reshape(T, K).T.reshape(-1)
    return pos_kt, row_gate, start, enblk


def _sc_dispatch_body(x_hbm, pos_hbm, xs_hbm, i0_v, i1_v, rows_v, s0, s1):
    wid = lax.axis_index("c") * NS + lax.axis_index("s")
    base = wid * TPW
    pltpu.sync_copy(pos_hbm.at[pl.ds(base, TPW)], i0_v)
    pltpu.sync_copy(pos_hbm.at[pl.ds(T + base, TPW)], i1_v)
    pltpu.sync_copy(x_hbm.at[pl.ds(base, TPW)], rows_v)
    cp0 = pltpu.async_copy(rows_v, xs_hbm.at[i0_v], s0)
    cp1 = pltpu.async_copy(rows_v, xs_hbm.at[i1_v], s1)
    cp0.wait()
    cp1.wait()


def _sc_dispatch(x, pos_kt):
    mesh = plsc.VectorSubcoreMesh(core_axis_name="c", subcore_axis_name="s")
    fn = functools.partial(
        pl.kernel,
        mesh=mesh,
        out_type=jax.ShapeDtypeStruct((R_PAD, D), jnp.float32),
        scratch_types=[
            pltpu.VMEM((TPW,), jnp.int32),
            pltpu.VMEM((TPW,), jnp.int32),
            pltpu.VMEM((TPW, D), jnp.float32),
            pltpu.SemaphoreType.DMA,
            pltpu.SemaphoreType.DMA,
        ],
    )(_sc_dispatch_body)
    return fn(x, pos_kt)


BLK_F = 512
NF = F // BLK_F


def _gemm_body(estart_ref, enblk_ref, xs_ref, w1_ref, w2_ref, g_ref, out_ref):
    e = pl.program_id(0)
    f = pl.program_id(1)
    s = estart_ref[e]
    n = enblk_ref[e]
    w1 = w1_ref[...]
    w2 = w2_ref[...]

    def blk(i, _):
        base = pl.multiple_of(s + i * PAD_U, PAD_U)
        rr = [pl.ds(base, BLK_R)]
        hs = [lax.dot_general(xs_ref[r, :], w1, (((1,), (1,)), ((), ())),
                              preferred_element_type=jnp.float32)
              for r in rr]
        ps = [lax.dot_general(h * jax.nn.sigmoid(h),
                              w2, (((1,), (1,)), ((), ())),
                              preferred_element_type=jnp.float32)
              for h in hs]

        @pl.when(f == 0)
        def _():
            for r, p in zip(rr, ps):
                out_ref[r, :] = p

        @pl.when((f > 0) & (f < NF - 1))
        def _():
            for r, p in zip(rr, ps):
                out_ref[r, :] = out_ref[r, :] + p

        @pl.when(f == NF - 1)
        def _():
            for r, p in zip(rr, ps):
                out_ref[r, :] = (out_ref[r, :] + p) * g_ref[r, :]

        return 0

    lax.fori_loop(0, n, blk, 0)


def _grouped_gemm(estart, enblk, xs, W1, W2, row_gate):
    grid_spec = pltpu.PrefetchScalarGridSpec(
        num_scalar_prefetch=2,
        grid=(E, NF),
        in_specs=[
            pl.BlockSpec((R_PAD, D), lambda e, f, es, en: (0, 0)),
            pl.BlockSpec((BLK_F, D), lambda e, f, es, en: (e * NF + f, 0)),
            pl.BlockSpec((D, BLK_F), lambda e, f, es, en: (e, f)),
            pl.BlockSpec((R_PAD, 1), lambda e, f, es, en: (0, 0)),
        ],
        out_specs=pl.BlockSpec((R_PAD, D), lambda e, f, es, en: (0, 0)),
    )
    return pl.pallas_call(
        _gemm_body,
        grid_spec=grid_spec,
        out_shape=jax.ShapeDtypeStruct((R_PAD, D), jnp.float32),
        compiler_params=pltpu.CompilerParams(
            dimension_semantics=("arbitrary", "arbitrary"),
            vmem_limit_bytes=63 * 1024 * 1024),
    )(estart, enblk, xs, W1.reshape(E * F, D), W2.reshape(E * D, F),
      row_gate[:, None])


def _sc_combine_body(ys_hbm, pos_hbm, out_hbm, i0_v, i1_v, r0_v, r1_v, sem):
    wid = lax.axis_index("c") * NS + lax.axis_index("s")
    base = wid * TPW
    for c in range(TPW // CCH):
        off = base + c * CCH
        pltpu.sync_copy(pos_hbm.at[pl.ds(off, CCH)], i0_v)
        pltpu.sync_copy(pos_hbm.at[pl.ds(T + off, CCH)], i1_v)
        cp0 = pltpu.async_copy(ys_hbm.at[i0_v], r0_v, sem)
        cp1 = pltpu.async_copy(ys_hbm.at[i1_v], r1_v, sem)
        cp0.wait()
        cp1.wait()

        def body(i, _):
            for j in range(D // LANES):
                sl = pl.ds(j * LANES, LANES)
                r0_v[i, sl] = r0_v[i, sl] + r1_v[i, sl]
            return 0

        lax.fori_loop(0, CCH, body, 0)
        pltpu.sync_copy(r0_v, out_hbm.at[pl.ds(off, CCH)])


def _sc_combine(ys, pos_kt):
    mesh = plsc.VectorSubcoreMesh(core_axis_name="c", subcore_axis_name="s")
    fn = functools.partial(
        pl.kernel,
        mesh=mesh,
        out_type=jax.ShapeDtypeStruct((T, D), jnp.float32),
        scratch_types=[
            pltpu.VMEM((CCH,), jnp.int32),
            pltpu.VMEM((CCH,), jnp.int32),
            pltpu.VMEM((CCH, D), jnp.float32),
            pltpu.VMEM((CCH, D), jnp.float32),
            pltpu.SemaphoreType.DMA,
        ],
    )(_sc_combine_body)
    return fn(ys, pos_kt)


def kernel(x, Wr, W1, W2):
    e1, e2, g1, g2 = _router(x, Wr)
    pos_kt, row_gate, estart, enblk = _route_metadata(e1, e2, g1, g2)
    xs = _sc_dispatch(x, pos_kt)
    ys = _grouped_gemm(estart, enblk, xs, W1, W2, row_gate)
    return _sc_combine(ys, pos_kt)


# speedup vs baseline: 1.1745x; 1.1115x over previous
import functools

import jax
import jax.numpy as jnp
from jax import lax
from jax.experimental import pallas as pl
from jax.experimental.pallas import tpu as pltpu
from jax.experimental.pallas import tpu_sc as plsc

T, D, F, E, K = 2048, 1024, 4096, 8, 2

BLK_R = 128
PAD_U = BLK_R
R_PAD = T * K + E * PAD_U

NC, NS = 2, 16
NW = NC * NS
TPW = T // NW
CCH = 32
LANES = 16


def _router_body(x_ref, wr_ref, e1_ref, e2_ref, g1_ref, g2_ref):
    logits = jnp.dot(x_ref[...], wr_ref[...],
                     preferred_element_type=jnp.float32)
    cols = lax.broadcasted_iota(jnp.int32, logits.shape, 1)
    l1 = jnp.max(logits, axis=1, keepdims=True)
    i1 = jnp.min(jnp.where(logits == l1, cols, E), axis=1, keepdims=True)
    masked = jnp.where(cols == i1, -jnp.inf, logits)
    l2 = jnp.max(masked, axis=1, keepdims=True)
    i2 = jnp.min(jnp.where(masked == l2, cols, E), axis=1, keepdims=True)
    g1 = jax.nn.sigmoid(l1 - l2)
    e1_ref[...] = i1
    e2_ref[...] = i2
    g1_ref[...] = g1
    g2_ref[...] = 1.0 - g1


def _router(x, Wr):
    return pl.pallas_call(
        _router_body,
        out_shape=(
            jax.ShapeDtypeStruct((T, 1), jnp.int32),
            jax.ShapeDtypeStruct((T, 1), jnp.int32),
            jax.ShapeDtypeStruct((T, 1), jnp.float32),
            jax.ShapeDtypeStruct((T, 1), jnp.float32),
        ),
    )(x, Wr)


def _route_metadata(e1, e2, g1, g2):
    flat_e = jnp.concatenate([e1, e2], axis=1).reshape(-1)
    flat_g = jnp.concatenate([g1, g2], axis=1).reshape(-1)
    onehot = (flat_e[:, None] == jnp.arange(E, dtype=jnp.int32)[None, :])
    onehot = onehot.astype(jnp.int32)
    counts = jnp.sum(onehot, axis=0)
    padded = ((counts + PAD_U - 1) // PAD_U) * PAD_U
    start = jnp.concatenate([jnp.zeros((1,), jnp.int32),
                             jnp.cumsum(padded)[:-1].astype(jnp.int32)])
    rank = jnp.cumsum(onehot, axis=0) - onehot
    rank_a = jnp.sum(rank * onehot, axis=1)
    pos = start[flat_e] + rank_a
    row_gate = jnp.zeros((R_PAD,), jnp.float32).at[pos].set(flat_g)
    enblk = (padded // PAD_U).astype(jnp.int32)
    pos_kt = pos.reshape(T, K).T.reshape(-1)
    return pos_kt, row_gate, start, enblk


def _sc_dispatch_body(x_hbm, pos_hbm, xs_hbm, i0_v, i1_v, rows_v, s0, s1):
    wid = lax.axis_index("c") * NS + lax.axis_index("s")
    base = wid * TPW
    pltpu.sync_copy(pos_hbm.at[pl.ds(base, TPW)], i0_v)
    pltpu.sync_copy(pos_hbm.at[pl.ds(T + base, TPW)], i1_v)
    pltpu.sync_copy(x_hbm.at[pl.ds(base, TPW)], rows_v)
    cp0 = pltpu.async_copy(rows_v, xs_hbm.at[i0_v], s0)
    cp1 = pltpu.async_copy(rows_v, xs_hbm.at[i1_v], s1)
    cp0.wait()
    cp1.wait()


def _sc_dispatch(x, pos_kt):
    mesh = plsc.VectorSubcoreMesh(core_axis_name="c", subcore_axis_name="s")
    fn = functools.partial(
        pl.kernel,
        mesh=mesh,
        out_type=jax.ShapeDtypeStruct((R_PAD, D), jnp.float32),
        scratch_types=[
            pltpu.VMEM((TPW,), jnp.int32),
            pltpu.VMEM((TPW,), jnp.int32),
            pltpu.VMEM((TPW, D), jnp.float32),
            pltpu.SemaphoreType.DMA,
            pltpu.SemaphoreType.DMA,
        ],
    )(_sc_dispatch_body)
    return fn(x, pos_kt)


BLK_F = 512
NF = F // BLK_F


def _gemm_body(estart_ref, enblk_ref, xs_ref, w1_ref, w2_ref, g_ref, out_ref):
    e = pl.program_id(0)
    f = pl.program_id(1)
    s = estart_ref[e]
    n = enblk_ref[e]
    w1 = w1_ref[...]
    w2 = w2_ref[...]

    def run(rr):
        hs = [lax.dot_general(xs_ref[r, :], w1, (((1,), (1,)), ((), ())),
                              preferred_element_type=jnp.float32)
              for r in rr]
        ps = [lax.dot_general(h * jax.nn.sigmoid(h),
                              w2, (((1,), (1,)), ((), ())),
                              preferred_element_type=jnp.float32)
              for h in hs]

        @pl.when(f == 0)
        def _():
            for r, p in zip(rr, ps):
                out_ref[r, :] = p

        @pl.when((f > 0) & (f < NF - 1))
        def _():
            for r, p in zip(rr, ps):
                out_ref[r, :] = out_ref[r, :] + p

        @pl.when(f == NF - 1)
        def _():
            for r, p in zip(rr, ps):
                out_ref[r, :] = (out_ref[r, :] + p) * g_ref[r, :]

    def pair(i, _):
        base = pl.multiple_of(s + i * 2 * PAD_U, PAD_U)
        run([pl.ds(base, BLK_R),
             pl.ds(pl.multiple_of(base + BLK_R, BLK_R), BLK_R)])
        return 0

    lax.fori_loop(0, n // 2, pair, 0)

    @pl.when(n % 2 == 1)
    def _():
        run([pl.ds(pl.multiple_of(s + (n - 1) * PAD_U, PAD_U), BLK_R)])


def _grouped_gemm(estart, enblk, xs, W1, W2, row_gate):
    grid_spec = pltpu.PrefetchScalarGridSpec(
        num_scalar_prefetch=2,
        grid=(E, NF),
        in_specs=[
            pl.BlockSpec((R_PAD, D), lambda e, f, es, en: (0, 0)),
            pl.BlockSpec((BLK_F, D), lambda e, f, es, en: (e * NF + f, 0)),
            pl.BlockSpec((D, BLK_F), lambda e, f, es, en: (e, f)),
            pl.BlockSpec((R_PAD, 1), lambda e, f, es, en: (0, 0)),
        ],
        out_specs=pl.BlockSpec((R_PAD, D), lambda e, f, es, en: (0, 0)),
    )
    return pl.pallas_call(
        _gemm_body,
        grid_spec=grid_spec,
        out_shape=jax.ShapeDtypeStruct((R_PAD, D), jnp.float32),
        compiler_params=pltpu.CompilerParams(
            dimension_semantics=("arbitrary", "arbitrary"),
            vmem_limit_bytes=63 * 1024 * 1024),
    )(estart, enblk, xs, W1.reshape(E * F, D), W2.reshape(E * D, F),
      row_gate[:, None])


def _sc_combine_body(ys_hbm, pos_hbm, out_hbm, i0_v, i1_v, r0_v, r1_v, sem):
    wid = lax.axis_index("c") * NS + lax.axis_index("s")
    base = wid * TPW
    for c in range(TPW // CCH):
        off = base + c * CCH
        pltpu.sync_copy(pos_hbm.at[pl.ds(off, CCH)], i0_v)
        pltpu.sync_copy(pos_hbm.at[pl.ds(T + off, CCH)], i1_v)
        cp0 = pltpu.async_copy(ys_hbm.at[i0_v], r0_v, sem)
        cp1 = pltpu.async_copy(ys_hbm.at[i1_v], r1_v, sem)
        cp0.wait()
        cp1.wait()

        def body(i, _):
            for j in range(D // LANES):
                sl = pl.ds(j * LANES, LANES)
                r0_v[i, sl] = r0_v[i, sl] + r1_v[i, sl]
            return 0

        lax.fori_loop(0, CCH, body, 0)
        pltpu.sync_copy(r0_v, out_hbm.at[pl.ds(off, CCH)])


def _sc_combine(ys, pos_kt):
    mesh = plsc.VectorSubcoreMesh(core_axis_name="c", subcore_axis_name="s")
    fn = functools.partial(
        pl.kernel,
        mesh=mesh,
        out_type=jax.ShapeDtypeStruct((T, D), jnp.float32),
        scratch_types=[
            pltpu.VMEM((CCH,), jnp.int32),
            pltpu.VMEM((CCH,), jnp.int32),
            pltpu.VMEM((CCH, D), jnp.float32),
            pltpu.VMEM((CCH, D), jnp.float32),
            pltpu.SemaphoreType.DMA,
        ],
    )(_sc_combine_body)
    return fn(ys, pos_kt)


def kernel(x, Wr, W1, W2):
    e1, e2, g1, g2 = _router(x, Wr)
    pos_kt, row_gate, estart, enblk = _route_metadata(e1, e2, g1, g2)
    xs = _sc_dispatch(x, pos_kt)
    ys = _grouped_gemm(estart, enblk, xs, W1, W2, row_gate)
    return _sc_combine(ys, pos_kt)


# speedup vs baseline: 1.2179x; 1.0370x over previous
import functools

import jax
import jax.numpy as jnp
from jax import lax
from jax.experimental import pallas as pl
from jax.experimental.pallas import tpu as pltpu
from jax.experimental.pallas import tpu_sc as plsc

T, D, F, E, K = 2048, 1024, 4096, 8, 2

BLK_R = 128
PAD_U = BLK_R
R_PAD = T * K + E * PAD_U

NC, NS = 2, 16
NW = NC * NS
TPW = T // NW
CCH = 32
LANES = 16


def _router_body(x_ref, wr_ref, e1_ref, e2_ref, g1_ref, g2_ref):
    logits = jnp.dot(x_ref[...], wr_ref[...],
                     preferred_element_type=jnp.float32)
    cols = lax.broadcasted_iota(jnp.int32, logits.shape, 1)
    l1 = jnp.max(logits, axis=1, keepdims=True)
    i1 = jnp.min(jnp.where(logits == l1, cols, E), axis=1, keepdims=True)
    masked = jnp.where(cols == i1, -jnp.inf, logits)
    l2 = jnp.max(masked, axis=1, keepdims=True)
    i2 = jnp.min(jnp.where(masked == l2, cols, E), axis=1, keepdims=True)
    g1 = jax.nn.sigmoid(l1 - l2)
    e1_ref[...] = i1
    e2_ref[...] = i2
    g1_ref[...] = g1
    g2_ref[...] = 1.0 - g1


def _router(x, Wr):
    return pl.pallas_call(
        _router_body,
        out_shape=(
            jax.ShapeDtypeStruct((T, 1), jnp.int32),
            jax.ShapeDtypeStruct((T, 1), jnp.int32),
            jax.ShapeDtypeStruct((T, 1), jnp.float32),
            jax.ShapeDtypeStruct((T, 1), jnp.float32),
        ),
    )(x, Wr)


def _route_metadata(e1, e2, g1, g2):
    flat_e = jnp.concatenate([e1, e2], axis=1).reshape(-1)
    flat_g = jnp.concatenate([g1, g2], axis=1).reshape(-1)
    onehot = (flat_e[:, None] == jnp.arange(E, dtype=jnp.int32)[None, :])
    onehot = onehot.astype(jnp.int32)
    counts = jnp.sum(onehot, axis=0)
    padded = ((counts + PAD_U - 1) // PAD_U) * PAD_U
    start = jnp.concatenate([jnp.zeros((1,), jnp.int32),
                             jnp.cumsum(padded)[:-1].astype(jnp.int32)])
    rank = jnp.cumsum(onehot, axis=0) - onehot
    rank_a = jnp.sum(rank * onehot, axis=1)
    pos = start[flat_e] + rank_a
    row_gate = jnp.zeros((R_PAD,), jnp.float32).at[pos].set(flat_g)
    enblk = (padded // PAD_U).astype(jnp.int32)
    pos_kt = pos.reshape(T, K).T.reshape(-1)
    return pos_kt, row_gate, start, enblk


def _sc_dispatch_body(x_hbm, pos_hbm, xs_hbm, i0_v, i1_v, rows_v, s0, s1):
    wid = lax.axis_index("c") * NS + lax.axis_index("s")
    base = wid * TPW
    pltpu.sync_copy(pos_hbm.at[pl.ds(base, TPW)], i0_v)
    pltpu.sync_copy(pos_hbm.at[pl.ds(T + base, TPW)], i1_v)
    pltpu.sync_copy(x_hbm.at[pl.ds(base, TPW)], rows_v)
    cp0 = pltpu.async_copy(rows_v, xs_hbm.at[i0_v], s0)
    cp1 = pltpu.async_copy(rows_v, xs_hbm.at[i1_v], s1)
    cp0.wait()
    cp1.wait()


def _sc_dispatch(x, pos_kt):
    mesh = plsc.VectorSubcoreMesh(core_axis_name="c", subcore_axis_name="s")
    fn = functools.partial(
        pl.kernel,
        mesh=mesh,
        out_type=jax.ShapeDtypeStruct((R_PAD, D), jnp.float32),
        scratch_types=[
            pltpu.VMEM((TPW,), jnp.int32),
            pltpu.VMEM((TPW,), jnp.int32),
            pltpu.VMEM((TPW, D), jnp.float32),
            pltpu.SemaphoreType.DMA,
            pltpu.SemaphoreType.DMA,
        ],
    )(_sc_dispatch_body)
    return fn(x, pos_kt)


BLK_F = 512
NF = F // BLK_F


def _gemm_body(estart_ref, enblk_ref, xs_ref, w1_ref, w2_ref, g_ref, out_ref):
    e = pl.program_id(0)
    f = pl.program_id(1)
    s = estart_ref[e]
    n = enblk_ref[e]
    w1 = w1_ref[...]
    w2 = w2_ref[...]

    def run(rr):
        hs = [lax.dot_general(xs_ref[r, :], w1, (((1,), (1,)), ((), ())),
                              preferred_element_type=jnp.float32)
              for r in rr]
        ps = [lax.dot_general(h * jax.nn.sigmoid(h),
                              w2, (((1,), (1,)), ((), ())),
                              preferred_element_type=jnp.float32)
              for h in hs]

        @pl.when(f == 0)
        def _():
            for r, p in zip(rr, ps):
                out_ref[r, :] = p

        @pl.when((f > 0) & (f < NF - 1))
        def _():
            for r, p in zip(rr, ps):
                out_ref[r, :] = out_ref[r, :] + p

        @pl.when(f == NF - 1)
        def _():
            for r, p in zip(rr, ps):
                out_ref[r, :] = (out_ref[r, :] + p) * g_ref[r, :]

    def quad(i, _):
        base = pl.multiple_of(s + i * 4 * PAD_U, PAD_U)
        run([pl.ds(pl.multiple_of(base + j * BLK_R, BLK_R), BLK_R)
             for j in range(4)])
        return 0

    lax.fori_loop(0, n // 4, quad, 0)
    rem = n % 4
    rbase = pl.multiple_of(s + (n - rem) * PAD_U, PAD_U)

    @pl.when(rem >= 2)
    def _():
        run([pl.ds(rbase, BLK_R),
             pl.ds(pl.multiple_of(rbase + BLK_R, BLK_R), BLK_R)])

    @pl.when(rem % 2 == 1)
    def _():
        run([pl.ds(pl.multiple_of(s + (n - 1) * PAD_U, PAD_U), BLK_R)])


def _grouped_gemm(estart, enblk, xs, W1, W2, row_gate):
    grid_spec = pltpu.PrefetchScalarGridSpec(
        num_scalar_prefetch=2,
        grid=(E, NF),
        in_specs=[
            pl.BlockSpec((R_PAD, D), lambda e, f, es, en: (0, 0)),
            pl.BlockSpec((BLK_F, D), lambda e, f, es, en: (e * NF + f, 0)),
            pl.BlockSpec((D, BLK_F), lambda e, f, es, en: (e, f)),
            pl.BlockSpec((R_PAD, 1), lambda e, f, es, en: (0, 0)),
        ],
        out_specs=pl.BlockSpec((R_PAD, D), lambda e, f, es, en: (0, 0)),
    )
    return pl.pallas_call(
        _gemm_body,
        grid_spec=grid_spec,
        out_shape=jax.ShapeDtypeStruct((R_PAD, D), jnp.float32),
        compiler_params=pltpu.CompilerParams(
            dimension_semantics=("arbitrary", "arbitrary"),
            vmem_limit_bytes=63 * 1024 * 1024),
    )(estart, enblk, xs, W1.reshape(E * F, D), W2.reshape(E * D, F),
      row_gate[:, None])


def _sc_combine_body(ys_hbm, pos_hbm, out_hbm, i0_v, i1_v, r0_v, r1_v, sem):
    wid = lax.axis_index("c") * NS + lax.axis_index("s")
    base = wid * TPW
    for c in range(TPW // CCH):
        off = base + c * CCH
        pltpu.sync_copy(pos_hbm.at[pl.ds(off, CCH)], i0_v)
        pltpu.sync_copy(pos_hbm.at[pl.ds(T + off, CCH)], i1_v)
        cp0 = pltpu.async_copy(ys_hbm.at[i0_v], r0_v, sem)
        cp1 = pltpu.async_copy(ys_hbm.at[i1_v], r1_v, sem)
        cp0.wait()
        cp1.wait()

        def body(i, _):
            for j in range(D // LANES):
                sl = pl.ds(j * LANES, LANES)
                r0_v[i, sl] = r0_v[i, sl] + r1_v[i, sl]
            return 0

        lax.fori_loop(0, CCH, body, 0)
        pltpu.sync_copy(r0_v, out_hbm.at[pl.ds(off, CCH)])


def _sc_combine(ys, pos_kt):
    mesh = plsc.VectorSubcoreMesh(core_axis_name="c", subcore_axis_name="s")
    fn = functools.partial(
        pl.kernel,
        mesh=mesh,
        out_type=jax.ShapeDtypeStruct((T, D), jnp.float32),
        scratch_types=[
            pltpu.VMEM((CCH,), jnp.int32),
            pltpu.VMEM((CCH,), jnp.int32),
            pltpu.VMEM((CCH, D), jnp.float32),
            pltpu.VMEM((CCH, D), jnp.float32),
            pltpu.SemaphoreType.DMA,
        ],
    )(_sc_combine_body)
    return fn(ys, pos_kt)


def kernel(x, Wr, W1, W2):
    e1, e2, g1, g2 = _router(x, Wr)
    pos_kt, row_gate, estart, enblk = _route_metadata(e1, e2, g1, g2)
    xs = _sc_dispatch(x, pos_kt)
    ys = _grouped_gemm(estart, enblk, xs, W1, W2, row_gate)
    return _sc_combine(ys, pos_kt)
